# baseline (speedup 1.0000x reference)
import functools

import jax
import jax.numpy as jnp
from jax import lax
from jax.experimental import pallas as pl
from jax.experimental.pallas import tpu as pltpu
import jax.experimental.pallas.tpu_sc as plsc

_N = 10000
_NPAD = 10240
_E = 320000
_G = 64
_BN_EPS = 1e-5


_K = 50
_NB = 8


@functools.partial(jax.jit, static_argnums=(2,))
def _sc_agg(h2, edge_index, Dh):
    info = plsc.get_sparse_core_info()
    NC, NS = info.num_cores, info.num_subcores
    EW = _E // NS
    K = _K
    NCHUNK = EW // K
    edges3 = edge_index.reshape(2 * NS, NCHUNK, K)
    NB = _NB
    NR = NCHUNK // NB
    RPS = _NPAD // NS
    ZR = 128

    mesh = plsc.VectorSubcoreMesh(core_axis_name="c", subcore_axis_name="s")

    @functools.partial(
        pl.kernel,
        out_type=jax.ShapeDtypeStruct((NC, _NPAD, Dh), jnp.float32),
        mesh=mesh,
        compiler_params=pltpu.CompilerParams(use_tc_tiling_on_sc=False),
        scratch_types=[
            pltpu.VMEM((NCHUNK, K), jnp.int32),
            pltpu.VMEM((NCHUNK, K), jnp.int32),
            pltpu.VMEM((NB, K, Dh), jnp.float32),
            pltpu.VMEM((ZR, Dh), jnp.float32),
            pltpu.VMEM_SHARED((_NPAD, Dh), jnp.float32),
            pltpu.SemaphoreType.DMA((NB,)),
            pltpu.SemaphoreType.DMA((NB,)),
        ],
    )
    def agg_kernel(h_hbm, e_hbm, out_hbm, sidx, didx, bufs, zbuf,
                   acc, gsem, ssem):
        c = lax.axis_index("c")
        s = lax.axis_index("s")

        zv = jnp.zeros((16,), jnp.float32)

        def zrow(i, _):
            r = i // (Dh // 16)
            j = i % (Dh // 16)
            zbuf[r, pl.ds(j * 16, 16)] = zv
            return 0

        lax.fori_loop(0, ZR * (Dh // 16), zrow, 0)

        def zcopy(i, _):
            pltpu.sync_copy(zbuf, acc.at[pl.ds(s * RPS + i * ZR, ZR)])
            return 0

        lax.fori_loop(0, RPS // ZR, zcopy, 0)

        pltpu.sync_copy(e_hbm.at[s], sidx)
        pltpu.sync_copy(e_hbm.at[NS + s], didx)
        plsc.subcore_barrier()

        def g_start(i, b):
            pltpu.async_copy(h_hbm.at[c].at[sidx.at[i]], bufs.at[b],
                             gsem.at[b])

        def g_wait(i, b):
            pltpu.make_async_copy(h_hbm.at[c].at[sidx.at[i]], bufs.at[b],
                                  gsem.at[b]).wait()

        def s_start(i, b):
            pltpu.async_copy(bufs.at[b], acc.at[didx.at[i]], ssem.at[b],
                             add=True)

        def s_wait(i, b):
            pltpu.make_async_copy(bufs.at[b], acc.at[didx.at[i]],
                                  ssem.at[b]).wait()

        for b in range(NB):
            g_start(b, b)

        def scatter_half(j):
            for b in range(NB):
                g_wait(j * NB + b, b)
                s_start(j * NB + b, b)

        def round_full(j, _):
            scatter_half(j)
            for b in range(NB):
                s_wait(j * NB + b, b)
                g_start((j + 1) * NB + b, b)
            return 0

        lax.fori_loop(0, NR - 1, round_full, 0)
        scatter_half(NR - 1)
        for b in range(NB):
            s_wait((NR - 1) * NB + b, b)
        plsc.subcore_barrier()

        pltpu.sync_copy(acc.at[pl.ds(s * RPS, RPS)],
                        out_hbm.at[c].at[pl.ds(s * RPS, RPS)])

    return agg_kernel(h2, edges3)


def _mlp_pool_body(x_ref, agg_ref, b_ref, w1_ref, b1_ref, g_ref, be_ref,
                   w2_ref, b2_ref, h_ref, p_ref):
    i = pl.program_id(0)
    xx = jnp.concatenate([x_ref[0] + agg_ref[0], x_ref[1] + agg_ref[1]],
                         axis=1)
    h = jnp.dot(xx, w1_ref[...], preferred_element_type=jnp.float32)
    h = (h + b1_ref[...]) * g_ref[...] + be_ref[...]
    h = jnp.maximum(h, 0.0)
    h = jnp.dot(h, w2_ref[...], preferred_element_type=jnp.float32) + b2_ref[...]
    h = jnp.maximum(h, 0.0)
    Hh = h.shape[1] // 2
    h_ref[0] = h[:, :Hh]
    h_ref[1] = h[:, Hh:]
    pblk = lax.dot_general(b_ref[...], h, (((0,), (0,)), ((), ())),
                           preferred_element_type=jnp.float32)

    @pl.when(i == 0)
    def _():
        p_ref[...] = pblk

    @pl.when(i > 0)
    def _():
        p_ref[...] += pblk


@functools.partial(jax.jit, static_argnums=(9,))
def _mlp_pool(x2, agg, onehot, w1, b1, g, be, w2, b2, Dh):
    BN = 2000
    H = w1.shape[1]
    grid = _N // BN
    return pl.pallas_call(
        _mlp_pool_body,
        grid=(grid,),
        in_specs=[
            pl.BlockSpec((2, BN, Dh), lambda i: (0, i, 0)),
            pl.BlockSpec((2, BN, Dh), lambda i: (0, i, 0)),
            pl.BlockSpec((BN, _G), lambda i: (i, 0)),
            pl.BlockSpec((2 * Dh, H), lambda i: (0, 0)),
            pl.BlockSpec((1, H), lambda i: (0, 0)),
            pl.BlockSpec((1, H), lambda i: (0, 0)),
            pl.BlockSpec((1, H), lambda i: (0, 0)),
            pl.BlockSpec((H, H), lambda i: (0, 0)),
            pl.BlockSpec((1, H), lambda i: (0, 0)),
        ],
        out_specs=[
            pl.BlockSpec((2, BN, H // 2), lambda i: (0, i, 0)),
            pl.BlockSpec((_G, H), lambda i: (0, 0)),
        ],
        out_shape=[
            jax.ShapeDtypeStruct((2, _N, H // 2), jnp.float32),
            jax.ShapeDtypeStruct((_G, H), jnp.float32),
        ],
    )(x2, agg, onehot, w1, b1, g, be, w2, b2)


def _head_body(p1_ref, p2_ref, p3_ref, w1_ref, b1_ref, w2_ref, b2_ref, o_ref):
    h = jnp.concatenate([p1_ref[...], p2_ref[...], p3_ref[...]], axis=1)
    h = jnp.dot(h, w1_ref[...], preferred_element_type=jnp.float32) + b1_ref[...]
    h = jnp.maximum(h, 0.0)
    h = jnp.dot(h, w2_ref[...], preferred_element_type=jnp.float32) + b2_ref[...]
    m = jnp.max(h, axis=1, keepdims=True)
    lse = m + jnp.log(jnp.sum(jnp.exp(h - m), axis=1, keepdims=True))
    o_ref[...] = h - lse


@jax.jit
def _head(p1, p2, p3, w1, b1, w2, b2):
    return pl.pallas_call(
        _head_body,
        out_shape=jax.ShapeDtypeStruct((_G, 2), jnp.float32),
    )(p1, p2, p3, w1, b1, w2, b2)


def kernel(x, edge_index, batch,
           c1_w1, c1_b1, c1_g, c1_be, c1_w2, c1_b2,
           c2_w1, c2_b1, c2_g, c2_be, c2_w2, c2_b2,
           c3_w1, c3_b1, c3_g, c3_be, c3_w2, c3_b2,
           lin1_w, lin1_b, lin2_w, lin2_b):
    onehot = (batch[:, None] == jnp.arange(_G, dtype=batch.dtype)[None, :]
              ).astype(jnp.float32)
    inv = 1.0 / jnp.sqrt(1.0 + _BN_EPS)

    def vec(v):
        return v.reshape(1, -1)

    x2 = x.reshape(_N, 2, 64).transpose(1, 0, 2)
    agg1 = _sc_agg(x2, edge_index, 64)
    h1, p1 = _mlp_pool(x2, agg1, onehot, c1_w1, vec(c1_b1), vec(c1_g * inv),
                       vec(c1_be), c1_w2, vec(c1_b2), 64)
    agg2 = _sc_agg(h1, edge_index, 32)
    h2, p2 = _mlp_pool(h1, agg2, onehot, c2_w1, vec(c2_b1), vec(c2_g * inv),
                       vec(c2_be), c2_w2, vec(c2_b2), 32)
    agg3 = _sc_agg(h2, edge_index, 32)
    h3, p3 = _mlp_pool(h2, agg3, onehot, c3_w1, vec(c3_b1), vec(c3_g * inv),
                       vec(c3_be), c3_w2, vec(c3_b2), 32)
    return _head(p1, p2, p3, lin1_w, vec(lin1_b), lin2_w, vec(lin2_b))


# speedup vs baseline: 1.1804x; 1.1804x over previous
import functools

import jax
import jax.numpy as jnp
from jax import lax
from jax.experimental import pallas as pl
from jax.experimental.pallas import tpu as pltpu
import jax.experimental.pallas.tpu_sc as plsc

_N = 10000
_NPAD = 10240
_E = 320000
_G = 64
_BN_EPS = 1e-5


_K = 50
_NB = 8


@functools.partial(jax.jit, static_argnums=(2,))
def _sc_agg(h2, edge_index, Dh):
    info = plsc.get_sparse_core_info()
    NC, NS = info.num_cores, info.num_subcores
    EW = _E // NS
    K = _K
    NCHUNK = EW // K
    edges3 = edge_index.reshape(2 * NS, NCHUNK, K)
    NB = _NB
    NR = NCHUNK // NB
    RPS = _NPAD // NS
    ZR = 128

    mesh = plsc.VectorSubcoreMesh(core_axis_name="c", subcore_axis_name="s")

    @functools.partial(
        pl.kernel,
        out_type=jax.ShapeDtypeStruct((NC, _NPAD, Dh), jnp.bfloat16),
        mesh=mesh,
        compiler_params=pltpu.CompilerParams(use_tc_tiling_on_sc=False),
        scratch_types=[
            pltpu.VMEM((NCHUNK, K), jnp.int32),
            pltpu.VMEM((NCHUNK, K), jnp.int32),
            pltpu.VMEM((NB, K, Dh), jnp.bfloat16),
            pltpu.VMEM((ZR, Dh), jnp.bfloat16),
            pltpu.VMEM_SHARED((_NPAD, Dh), jnp.bfloat16),
            pltpu.SemaphoreType.DMA((NB,)),
            pltpu.SemaphoreType.DMA((NB,)),
        ],
    )
    def agg_kernel(h_hbm, e_hbm, out_hbm, sidx, didx, bufs, zbuf,
                   acc, gsem, ssem):
        c = lax.axis_index("c")
        s = lax.axis_index("s")

        zv = jnp.zeros((32,), jnp.bfloat16)

        def zrow(i, _):
            r = i // (Dh // 32)
            j = i % (Dh // 32)
            zbuf[r, pl.ds(j * 32, 32)] = zv
            return 0

        lax.fori_loop(0, ZR * (Dh // 32), zrow, 0)

        def zcopy(i, _):
            pltpu.sync_copy(zbuf, acc.at[pl.ds(s * RPS + i * ZR, ZR)])
            return 0

        lax.fori_loop(0, RPS // ZR, zcopy, 0)

        pltpu.sync_copy(e_hbm.at[s], sidx)
        pltpu.sync_copy(e_hbm.at[NS + s], didx)
        plsc.subcore_barrier()

        def g_start(i, b):
            pltpu.async_copy(h_hbm.at[c].at[sidx.at[i]], bufs.at[b],
                             gsem.at[b])

        def g_wait(i, b):
            pltpu.make_async_copy(h_hbm.at[c].at[sidx.at[i]], bufs.at[b],
                                  gsem.at[b]).wait()

        def s_start(i, b):
            pltpu.async_copy(bufs.at[b], acc.at[didx.at[i]], ssem.at[b],
                             add=True)

        def s_wait(i, b):
            pltpu.make_async_copy(bufs.at[b], acc.at[didx.at[i]],
                                  ssem.at[b]).wait()

        for b in range(NB):
            g_start(b, b)

        def scatter_half(j):
            for b in range(NB):
                g_wait(j * NB + b, b)
                s_start(j * NB + b, b)

        def round_full(j, _):
            scatter_half(j)
            for b in range(NB):
                s_wait(j * NB + b, b)
                g_start((j + 1) * NB + b, b)
            return 0

        lax.fori_loop(0, NR - 1, round_full, 0)
        scatter_half(NR - 1)
        for b in range(NB):
            s_wait((NR - 1) * NB + b, b)
        plsc.subcore_barrier()

        pltpu.sync_copy(acc.at[pl.ds(s * RPS, RPS)],
                        out_hbm.at[c].at[pl.ds(s * RPS, RPS)])

    return agg_kernel(h2, edges3)


def _mlp_pool_body(x_ref, agg_ref, b_ref, w1_ref, b1_ref, g_ref, be_ref,
                   w2_ref, b2_ref, h_ref, p_ref):
    i = pl.program_id(0)
    f32 = jnp.float32
    xx = jnp.concatenate(
        [x_ref[0].astype(f32) + agg_ref[0].astype(f32),
         x_ref[1].astype(f32) + agg_ref[1].astype(f32)], axis=1)
    h = jnp.dot(xx, w1_ref[...], preferred_element_type=jnp.float32)
    h = (h + b1_ref[...]) * g_ref[...] + be_ref[...]
    h = jnp.maximum(h, 0.0)
    h = jnp.dot(h, w2_ref[...], preferred_element_type=jnp.float32) + b2_ref[...]
    h = jnp.maximum(h, 0.0)
    Hh = h.shape[1] // 2
    hb = h.astype(jnp.bfloat16)
    h_ref[0] = hb[:, :Hh]
    h_ref[1] = hb[:, Hh:]
    pblk = lax.dot_general(b_ref[...], h, (((0,), (0,)), ((), ())),
                           preferred_element_type=jnp.float32)

    @pl.when(i == 0)
    def _():
        p_ref[...] = pblk

    @pl.when(i > 0)
    def _():
        p_ref[...] += pblk


@functools.partial(jax.jit, static_argnums=(9,))
def _mlp_pool(x2, agg, onehot, w1, b1, g, be, w2, b2, Dh):
    BN = 2000
    H = w1.shape[1]
    grid = _N // BN
    return pl.pallas_call(
        _mlp_pool_body,
        grid=(grid,),
        in_specs=[
            pl.BlockSpec((2, BN, Dh), lambda i: (0, i, 0)),
            pl.BlockSpec((2, BN, Dh), lambda i: (0, i, 0)),
            pl.BlockSpec((BN, _G), lambda i: (i, 0)),
            pl.BlockSpec((2 * Dh, H), lambda i: (0, 0)),
            pl.BlockSpec((1, H), lambda i: (0, 0)),
            pl.BlockSpec((1, H), lambda i: (0, 0)),
            pl.BlockSpec((1, H), lambda i: (0, 0)),
            pl.BlockSpec((H, H), lambda i: (0, 0)),
            pl.BlockSpec((1, H), lambda i: (0, 0)),
        ],
        out_specs=[
            pl.BlockSpec((2, BN, H // 2), lambda i: (0, i, 0)),
            pl.BlockSpec((_G, H), lambda i: (0, 0)),
        ],
        out_shape=[
            jax.ShapeDtypeStruct((2, _N, H // 2), jnp.bfloat16),
            jax.ShapeDtypeStruct((_G, H), jnp.float32),
        ],
    )(x2, agg, onehot, w1, b1, g, be, w2, b2)


def _head_body(p1_ref, p2_ref, p3_ref, w1_ref, b1_ref, w2_ref, b2_ref, o_ref):
    h = jnp.concatenate([p1_ref[...], p2_ref[...], p3_ref[...]], axis=1)
    h = jnp.dot(h, w1_ref[...], preferred_element_type=jnp.float32) + b1_ref[...]
    h = jnp.maximum(h, 0.0)
    h = jnp.dot(h, w2_ref[...], preferred_element_type=jnp.float32) + b2_ref[...]
    m = jnp.max(h, axis=1, keepdims=True)
    lse = m + jnp.log(jnp.sum(jnp.exp(h - m), axis=1, keepdims=True))
    o_ref[...] = h - lse


@jax.jit
def _head(p1, p2, p3, w1, b1, w2, b2):
    return pl.pallas_call(
        _head_body,
        out_shape=jax.ShapeDtypeStruct((_G, 2), jnp.float32),
    )(p1, p2, p3, w1, b1, w2, b2)


def kernel(x, edge_index, batch,
           c1_w1, c1_b1, c1_g, c1_be, c1_w2, c1_b2,
           c2_w1, c2_b1, c2_g, c2_be, c2_w2, c2_b2,
           c3_w1, c3_b1, c3_g, c3_be, c3_w2, c3_b2,
           lin1_w, lin1_b, lin2_w, lin2_b):
    onehot = (batch[:, None] == jnp.arange(_G, dtype=batch.dtype)[None, :]
              ).astype(jnp.float32)
    inv = 1.0 / jnp.sqrt(1.0 + _BN_EPS)

    def vec(v):
        return v.reshape(1, -1)

    x2 = x.reshape(_N, 2, 64).transpose(1, 0, 2)
    agg1 = _sc_agg(x2.astype(jnp.bfloat16), edge_index, 64)
    h1, p1 = _mlp_pool(x2, agg1, onehot, c1_w1, vec(c1_b1), vec(c1_g * inv),
                       vec(c1_be), c1_w2, vec(c1_b2), 64)
    agg2 = _sc_agg(h1, edge_index, 32)
    h2, p2 = _mlp_pool(h1, agg2, onehot, c2_w1, vec(c2_b1), vec(c2_g * inv),
                       vec(c2_be), c2_w2, vec(c2_b2), 32)
    agg3 = _sc_agg(h2, edge_index, 32)
    h3, p3 = _mlp_pool(h2, agg3, onehot, c3_w1, vec(c3_b1), vec(c3_g * inv),
                       vec(c3_be), c3_w2, vec(c3_b2), 32)
    return _head(p1, p2, p3, lin1_w, vec(lin1_b), lin2_w, vec(lin2_b))


# speedup vs baseline: 1.4030x; 1.1886x over previous
import functools

import jax
import jax.numpy as jnp
from jax import lax
from jax.experimental import pallas as pl
from jax.experimental.pallas import tpu as pltpu
import jax.experimental.pallas.tpu_sc as plsc

_N = 10000
_NPAD = 10240
_E = 320000
_G = 64
_BN_EPS = 1e-5


_K = 100
_NB = 8


@functools.partial(jax.jit, static_argnums=(2,))
def _sc_agg(h2, edge_index, Dh):
    info = plsc.get_sparse_core_info()
    NC, NS = info.num_cores, info.num_subcores
    EW = _E // NS
    K = _K
    NCHUNK = EW // K
    edges3 = edge_index.reshape(2 * NS, NCHUNK, K)
    NB = _NB
    NR = NCHUNK // NB
    RPS = _NPAD // NS
    ZR = 128

    mesh = plsc.VectorSubcoreMesh(core_axis_name="c", subcore_axis_name="s")

    @functools.partial(
        pl.kernel,
        out_type=jax.ShapeDtypeStruct((NC, _NPAD, Dh), jnp.bfloat16),
        mesh=mesh,
        compiler_params=pltpu.CompilerParams(use_tc_tiling_on_sc=False),
        scratch_types=[
            pltpu.VMEM((NCHUNK, K), jnp.int32),
            pltpu.VMEM((NCHUNK, K), jnp.int32),
            pltpu.VMEM((NB, K, Dh), jnp.bfloat16),
            pltpu.VMEM((ZR, Dh), jnp.bfloat16),
            pltpu.VMEM_SHARED((_NPAD, Dh), jnp.bfloat16),
            pltpu.SemaphoreType.DMA((NB,)),
            pltpu.SemaphoreType.DMA((NB,)),
        ],
    )
    def agg_kernel(h_hbm, e_hbm, out_hbm, sidx, didx, bufs, zbuf,
                   acc, gsem, ssem):
        c = lax.axis_index("c")
        s = lax.axis_index("s")

        zv = jnp.zeros((32,), jnp.bfloat16)

        def zrow(i, _):
            r = i // (Dh // 32)
            j = i % (Dh // 32)
            zbuf[r, pl.ds(j * 32, 32)] = zv
            return 0

        lax.fori_loop(0, ZR * (Dh // 32), zrow, 0)

        def zcopy(i, _):
            pltpu.sync_copy(zbuf, acc.at[pl.ds(s * RPS + i * ZR, ZR)])
            return 0

        lax.fori_loop(0, RPS // ZR, zcopy, 0)

        pltpu.sync_copy(e_hbm.at[s], sidx)
        pltpu.sync_copy(e_hbm.at[NS + s], didx)
        plsc.subcore_barrier()

        def g_start(i, b):
            pltpu.async_copy(h_hbm.at[c].at[sidx.at[i]], bufs.at[b],
                             gsem.at[b])

        def g_wait(i, b):
            pltpu.make_async_copy(h_hbm.at[c].at[sidx.at[i]], bufs.at[b],
                                  gsem.at[b]).wait()

        def s_start(i, b):
            pltpu.async_copy(bufs.at[b], acc.at[didx.at[i]], ssem.at[b],
                             add=True)

        def s_wait(i, b):
            pltpu.make_async_copy(bufs.at[b], acc.at[didx.at[i]],
                                  ssem.at[b]).wait()

        for b in range(NB):
            g_start(b, b)

        def scatter_half(j):
            for b in range(NB):
                g_wait(j * NB + b, b)
                s_start(j * NB + b, b)

        def round_full(j, _):
            scatter_half(j)
            for b in range(NB):
                s_wait(j * NB + b, b)
                g_start((j + 1) * NB + b, b)
            return 0

        lax.fori_loop(0, NR - 1, round_full, 0)
        scatter_half(NR - 1)
        for b in range(NB):
            s_wait((NR - 1) * NB + b, b)
        plsc.subcore_barrier()

        pltpu.sync_copy(acc.at[pl.ds(s * RPS, RPS)],
                        out_hbm.at[c].at[pl.ds(s * RPS, RPS)])

    return agg_kernel(h2, edges3)


def _mlp_pool_body(x_ref, agg_ref, b_ref, w1_ref, b1_ref, g_ref, be_ref,
                   w2_ref, b2_ref, h_ref, p_ref):
    i = pl.program_id(0)
    f32 = jnp.float32
    xx = jnp.concatenate(
        [x_ref[0].astype(f32) + agg_ref[0].astype(f32),
         x_ref[1].astype(f32) + agg_ref[1].astype(f32)], axis=1)
    h = jnp.dot(xx, w1_ref[...], preferred_element_type=jnp.float32)
    h = (h + b1_ref[...]) * g_ref[...] + be_ref[...]
    h = jnp.maximum(h, 0.0)
    h = jnp.dot(h, w2_ref[...], preferred_element_type=jnp.float32) + b2_ref[...]
    h = jnp.maximum(h, 0.0)
    Hh = h.shape[1] // 2
    hb = h.astype(jnp.bfloat16)
    h_ref[0] = hb[:, :Hh]
    h_ref[1] = hb[:, Hh:]
    pblk = lax.dot_general(b_ref[...], hb, (((0,), (0,)), ((), ())),
                           preferred_element_type=jnp.float32)

    @pl.when(i == 0)
    def _():
        p_ref[...] = pblk

    @pl.when(i > 0)
    def _():
        p_ref[...] += pblk


@functools.partial(jax.jit, static_argnums=(9,))
def _mlp_pool(x2, agg, onehot, w1, b1, g, be, w2, b2, Dh):
    BN = 2000
    H = w1.shape[1]
    grid = _N // BN
    return pl.pallas_call(
        _mlp_pool_body,
        grid=(grid,),
        in_specs=[
            pl.BlockSpec((2, BN, Dh), lambda i: (0, i, 0)),
            pl.BlockSpec((2, BN, Dh), lambda i: (0, i, 0)),
            pl.BlockSpec((BN, _G), lambda i: (i, 0)),
            pl.BlockSpec((2 * Dh, H), lambda i: (0, 0)),
            pl.BlockSpec((1, H), lambda i: (0, 0)),
            pl.BlockSpec((1, H), lambda i: (0, 0)),
            pl.BlockSpec((1, H), lambda i: (0, 0)),
            pl.BlockSpec((H, H), lambda i: (0, 0)),
            pl.BlockSpec((1, H), lambda i: (0, 0)),
        ],
        out_specs=[
            pl.BlockSpec((2, BN, H // 2), lambda i: (0, i, 0)),
            pl.BlockSpec((_G, H), lambda i: (0, 0)),
        ],
        out_shape=[
            jax.ShapeDtypeStruct((2, _N, H // 2), jnp.bfloat16),
            jax.ShapeDtypeStruct((_G, H), jnp.float32),
        ],
    )(x2, agg, onehot, w1, b1, g, be, w2, b2)


def _head_body(p1_ref, p2_ref, p3_ref, w1_ref, b1_ref, w2_ref, b2_ref, o_ref):
    h = jnp.concatenate([p1_ref[...], p2_ref[...], p3_ref[...]], axis=1)
    h = jnp.dot(h, w1_ref[...], preferred_element_type=jnp.float32) + b1_ref[...]
    h = jnp.maximum(h, 0.0)
    h = jnp.dot(h, w2_ref[...], preferred_element_type=jnp.float32) + b2_ref[...]
    m = jnp.max(h, axis=1, keepdims=True)
    lse = m + jnp.log(jnp.sum(jnp.exp(h - m), axis=1, keepdims=True))
    o_ref[...] = h - lse


@jax.jit
def _head(p1, p2, p3, w1, b1, w2, b2):
    return pl.pallas_call(
        _head_body,
        out_shape=jax.ShapeDtypeStruct((_G, 2), jnp.float32),
    )(p1, p2, p3, w1, b1, w2, b2)


def kernel(x, edge_index, batch,
           c1_w1, c1_b1, c1_g, c1_be, c1_w2, c1_b2,
           c2_w1, c2_b1, c2_g, c2_be, c2_w2, c2_b2,
           c3_w1, c3_b1, c3_g, c3_be, c3_w2, c3_b2,
           lin1_w, lin1_b, lin2_w, lin2_b):
    onehot = (batch[:, None] == jnp.arange(_G, dtype=batch.dtype)[None, :]
              ).astype(jnp.bfloat16)
    inv = 1.0 / jnp.sqrt(1.0 + _BN_EPS)

    def vec(v):
        return v.reshape(1, -1)

    x2 = x.reshape(_N, 2, 64).transpose(1, 0, 2)
    agg1 = _sc_agg(x2.astype(jnp.bfloat16), edge_index, 64)
    h1, p1 = _mlp_pool(x2, agg1, onehot, c1_w1, vec(c1_b1), vec(c1_g * inv),
                       vec(c1_be), c1_w2, vec(c1_b2), 64)
    agg2 = _sc_agg(h1, edge_index, 32)
    h2, p2 = _mlp_pool(h1, agg2, onehot, c2_w1, vec(c2_b1), vec(c2_g * inv),
                       vec(c2_be), c2_w2, vec(c2_b2), 32)
    agg3 = _sc_agg(h2, edge_index, 32)
    h3, p3 = _mlp_pool(h2, agg3, onehot, c3_w1, vec(c3_b1), vec(c3_g * inv),
                       vec(c3_be), c3_w2, vec(c3_b2), 32)
    return _head(p1, p2, p3, lin1_w, vec(lin1_b), lin2_w, vec(lin2_b))
